# Initial kernel scaffold; baseline (speedup 1.0000x reference)
import jax
import jax.numpy as jnp
from jax.experimental import pallas as pl


def kernel(x, pos, edge_index, W1, W2):
    raise NotImplementedError("write your pallas kernel here")


# speedup vs baseline: 1.4355x; 1.4355x over previous
import functools

import jax
import jax.numpy as jnp
import numpy as np
from jax import lax
from jax.experimental import pallas as pl
from jax.experimental.pallas import tpu as pltpu
from jax.experimental.pallas import tpu_sc as plsc

N_NODES = 10000
N_EDGES = 160000
D_F = 8
RADIUS = 5.0
NB = 10
ACT_NORM = 1.6790

NW = 32
CH = 128
CPW = 80
E_PAD = 163840
EB = 1024

_STEP = RADIUS / (NB + 1)
_EMB_C = 1.14136 * float(np.exp(2.0)) * float(np.sqrt(NB))

_A16 = np.zeros((16, 128), np.float32)
for _i in range(8):
    for _j in range(8):
        _A16[_i, _i * 8 + _j] = 1.0
_B16 = np.zeros((16, 128), np.float32)
for _i in range(9):
    for _j in range(8):
        _B16[_j, _i * 8 + _j] = 1.0
_RH = np.zeros((16, 128), np.float32)
for _m in range(16):
    for _k in range(8):
        _RH[_m, _m * 8 + _k] = 1.0
_S = np.zeros((128, 8), np.float32)
for _m in range(16):
    for _k in range(8):
        _S[_m * 8 + _k, _k] = 1.0


def _sc_gather(table, idx_all):
    mesh = plsc.VectorSubcoreMesh(core_axis_name="c", subcore_axis_name="s")
    n_idx = idx_all.shape[0]

    @functools.partial(
        pl.kernel,
        out_type=jax.ShapeDtypeStruct((n_idx, 16), jnp.float32),
        mesh=mesh,
        scratch_types=[
            pltpu.VMEM((CH,), jnp.int32),
            pltpu.VMEM((CH, 16), jnp.float32),
            pltpu.SemaphoreType.DMA,
        ],
        compiler_params=pltpu.CompilerParams(use_tc_tiling_on_sc=False),
    )
    def k(table_hbm, idx_hbm, out_hbm, idx_v, rows_v, sem):
        wid = lax.axis_index("s") * 2 + lax.axis_index("c")
        base = wid * (CPW * CH)

        def body(c, carry):
            off = pl.multiple_of(base + c * CH, CH)
            pltpu.sync_copy(idx_hbm.at[pl.ds(off, CH)], idx_v)
            pltpu.async_copy(table_hbm.at[idx_v], rows_v, sem).wait()
            pltpu.sync_copy(rows_v, out_hbm.at[pl.ds(off, CH)])
            return carry

        lax.fori_loop(0, CPW, body, 0)

    return k(table, idx_all)


def _tc_body(gl_ref, gr_ref, w1_ref, w2_ref, a_ref, b_ref, rh_ref, s_ref,
             out_ref):
    gl = gl_ref[...]
    gr = gr_ref[...]
    lane = lax.broadcasted_iota(jnp.int32, (EB, 16), 1)

    d = gl - gr
    d2 = d * d
    r2 = jnp.sum(jnp.where((lane >= 8) & (lane < 11), d2, 0.0), axis=1,
                 keepdims=True)
    r = jnp.sqrt(r2 + 1e-12)

    vals = (lane.astype(jnp.float32) + 1.0) * _STEP
    diff = (r - vals) * (1.0 / _STEP)
    z1 = diff + 1.0
    z2 = 1.0 - diff
    s1 = jnp.where(z1 > 0, jnp.exp(-1.0 / jnp.where(z1 > 0, z1, 1.0)), 0.0)
    s2 = jnp.where(z2 > 0, jnp.exp(-1.0 / jnp.where(z2 > 0, z2, 1.0)), 0.0)
    emb = jnp.where(lane < NB, _EMB_C * s1 * s2, 0.0)

    dot = functools.partial(jnp.dot, precision=lax.Precision.HIGHEST,
                            preferred_element_type=jnp.float32)
    h = ACT_NORM * jax.nn.silu(dot(emb, w1_ref[...]))
    hb = dot(h, rh_ref[...])

    xb1 = dot(gl, a_ref[...])
    l128 = lax.broadcasted_iota(jnp.int32, (EB, 128), 1)
    xb1 = jnp.where((l128 >= 64) & (l128 < 72), 1.0, xb1)
    xb2 = dot(gr, b_ref[...])
    g = xb1 * xb2

    p = dot(g, w2_ref[...])
    out_ref[...] = dot(p * hb, s_ref[...])


def _tc_compute(gl, gr, w1p, w2p, a16, b16, rh, s):
    grid = E_PAD // EB
    full = lambda i: (0, 0)
    tile = lambda i: (i, 0)
    return pl.pallas_call(
        _tc_body,
        grid=(grid,),
        in_specs=[
            pl.BlockSpec((EB, 16), tile),
            pl.BlockSpec((EB, 16), tile),
            pl.BlockSpec((16, 16), full),
            pl.BlockSpec((128, 128), full),
            pl.BlockSpec((16, 128), full),
            pl.BlockSpec((16, 128), full),
            pl.BlockSpec((16, 128), full),
            pl.BlockSpec((128, 8), full),
        ],
        out_specs=pl.BlockSpec((EB, 8), tile),
        out_shape=jax.ShapeDtypeStruct((E_PAD, 8), jnp.float32),
    )(gl, gr, w1p, w2p, a16, b16, rh, s)


def kernel(x, pos, edge_index, W1, W2):
    idx = edge_index.astype(jnp.int32)
    table = jnp.concatenate(
        [x, pos, jnp.zeros((N_NODES, 5), x.dtype)], axis=1)
    pad = E_PAD - N_EDGES
    lig = jnp.pad(idx[1], (0, pad))
    rec = jnp.pad(idx[0], (0, pad))
    idx_all = jnp.concatenate([lig, rec])

    g = _sc_gather(table, idx_all)
    gl, gr = g[:E_PAD], g[E_PAD:]

    w1p = jnp.concatenate(
        [W1, jnp.zeros((6, 16), W1.dtype)], axis=0) / np.sqrt(NB)
    w2r = W2.reshape(16, 9, 8, 8).transpose(1, 2, 0, 3).reshape(72, 128)
    w2p = jnp.concatenate(
        [w2r, jnp.zeros((56, 128), W2.dtype)], axis=0) / (4.0 * np.sqrt(72.0))

    out = _tc_compute(gl, gr, w1p, w2p,
                      jnp.asarray(_A16), jnp.asarray(_B16),
                      jnp.asarray(_RH), jnp.asarray(_S))
    return out[:N_EDGES]


# speedup vs baseline: 1.7946x; 1.2501x over previous
import functools

import jax
import jax.numpy as jnp
import numpy as np
from jax import lax
from jax.experimental import pallas as pl
from jax.experimental.pallas import tpu as pltpu
from jax.experimental.pallas import tpu_sc as plsc

N_NODES = 10000
N_EDGES = 160000
RADIUS = 5.0
NB = 10
ACT_NORM = 1.6790

NW = 32
CH = 128
CPW = 80
E_PAD = 163840
EB = 2048

_STEP = RADIUS / (NB + 1)
_ISTEP2 = 1.0 / (_STEP * _STEP)
_EMB_C = 1.14136 * float(np.exp(2.0)) * float(np.sqrt(NB))

_A16 = np.zeros((16, 128), np.float32)
for _i in range(8):
    for _j in range(8):
        _A16[_i, _i * 8 + _j] = 1.0
for _j in range(8):
    _A16[11, 64 + _j] = 1.0
_B16 = np.zeros((16, 128), np.float32)
for _i in range(9):
    for _j in range(8):
        _B16[_j, _i * 8 + _j] = 1.0
_RH = np.zeros((16, 128), np.float32)
for _m in range(16):
    for _k in range(8):
        _RH[_m, _m * 8 + _k] = 1.0
_S = np.zeros((128, 8), np.float32)
for _m in range(16):
    for _k in range(8):
        _S[_m * 8 + _k, _k] = 1.0


def _sc_gather(table, idx_all):
    mesh = plsc.VectorSubcoreMesh(core_axis_name="c", subcore_axis_name="s")
    n_idx = idx_all.shape[0]

    @functools.partial(
        pl.kernel,
        out_type=jax.ShapeDtypeStruct((n_idx, 16), jnp.float32),
        mesh=mesh,
        scratch_types=[
            pltpu.VMEM((CH,), jnp.int32),
            pltpu.VMEM((CH, 16), jnp.float32),
            pltpu.SemaphoreType.DMA,
        ],
        compiler_params=pltpu.CompilerParams(use_tc_tiling_on_sc=False),
    )
    def k(table_hbm, idx_hbm, out_hbm, idx_v, rows_v, sem):
        wid = lax.axis_index("s") * 2 + lax.axis_index("c")
        base = wid * (CPW * CH)

        def body(c, carry):
            off = pl.multiple_of(base + c * CH, CH)
            pltpu.sync_copy(idx_hbm.at[pl.ds(off, CH)], idx_v)
            pltpu.async_copy(table_hbm.at[idx_v], rows_v, sem).wait()
            pltpu.sync_copy(rows_v, out_hbm.at[pl.ds(off, CH)])
            return carry

        lax.fori_loop(0, CPW, body, 0)

    return k(table, idx_all)


def _tc_body(gl_ref, gr_ref, w1b_ref, w2_ref, a_ref, b_ref, s_ref, out_ref):
    gl = gl_ref[...]
    gr = gr_ref[...]
    lane = lax.broadcasted_iota(jnp.int32, (EB, 16), 1)

    d = gl - gr
    d2 = d * d
    r2 = jnp.sum(jnp.where((lane >= 8) & (lane < 11), d2, 0.0), axis=1,
                 keepdims=True)
    t = jnp.sqrt((r2 + 1e-12) * _ISTEP2)
    diff = t - (lane.astype(jnp.float32) + 1.0)
    q = diff * diff
    den = 1.0 - q
    emb = jnp.where(q < 1.0, jnp.exp(-2.0 / den), 0.0)

    dot = functools.partial(jnp.dot, precision=lax.Precision.HIGHEST,
                            preferred_element_type=jnp.float32)
    yb = dot(emb, w1b_ref[...])
    hb = ACT_NORM * jax.nn.silu(yb)

    xb1 = dot(gl, a_ref[...])
    xb2 = dot(gr, b_ref[...])
    g = xb1 * xb2

    p = dot(g, w2_ref[...])
    out_ref[...] = dot(p * hb, s_ref[...])


def _tc_compute(gl, gr, w1b, w2p, a16, b16, s):
    grid = E_PAD // EB
    full = lambda i: (0, 0)
    tile = lambda i: (i, 0)
    return pl.pallas_call(
        _tc_body,
        grid=(grid,),
        in_specs=[
            pl.BlockSpec((EB, 16), tile),
            pl.BlockSpec((EB, 16), tile),
            pl.BlockSpec((16, 128), full),
            pl.BlockSpec((128, 128), full),
            pl.BlockSpec((16, 128), full),
            pl.BlockSpec((16, 128), full),
            pl.BlockSpec((128, 8), full),
        ],
        out_specs=pl.BlockSpec((EB, 8), tile),
        out_shape=jax.ShapeDtypeStruct((E_PAD, 8), jnp.float32),
    )(gl, gr, w1b, w2p, a16, b16, s)


def kernel(x, pos, edge_index, W1, W2):
    idx = edge_index.astype(jnp.int32)
    table = jnp.concatenate(
        [x, pos, jnp.ones((N_NODES, 1), x.dtype),
         jnp.zeros((N_NODES, 4), x.dtype)], axis=1)
    pad = E_PAD - N_EDGES
    lig = jnp.pad(idx[1], (0, pad))
    rec = jnp.pad(idx[0], (0, pad))
    idx_all = jnp.concatenate([lig, rec])

    g = _sc_gather(table, idx_all)
    gl, gr = g[:E_PAD], g[E_PAD:]

    w1p = jnp.concatenate(
        [W1, jnp.zeros((6, 16), W1.dtype)], axis=0) * (_EMB_C / np.sqrt(NB))
    w1b = w1p @ jnp.asarray(_RH)
    w2r = W2.reshape(16, 9, 8, 8).transpose(1, 2, 0, 3).reshape(72, 128)
    w2p = jnp.concatenate(
        [w2r, jnp.zeros((56, 128), W2.dtype)], axis=0) / (4.0 * np.sqrt(72.0))

    out = _tc_compute(gl, gr, w1b, w2p,
                      jnp.asarray(_A16), jnp.asarray(_B16), jnp.asarray(_S))
    return out[:N_EDGES]


# speedup vs baseline: 1.9187x; 1.0692x over previous
import functools

import jax
import jax.numpy as jnp
import numpy as np
from jax import lax
from jax.experimental import pallas as pl
from jax.experimental.pallas import tpu as pltpu
from jax.experimental.pallas import tpu_sc as plsc

N_NODES = 10000
N_EDGES = 160000
RADIUS = 5.0
NB = 10
ACT_NORM = 1.6790

NW = 32
CH = 128
CPW = 80
E_PAD = 163840
EB = 2048

_STEP = RADIUS / (NB + 1)
_ISTEP2 = 1.0 / (_STEP * _STEP)
_EMB_C = 1.14136 * float(np.exp(2.0)) * float(np.sqrt(NB))

_A16 = np.zeros((16, 128), np.float32)
for _i in range(8):
    for _j in range(8):
        _A16[_i, _i * 8 + _j] = 1.0
for _j in range(8):
    _A16[11, 64 + _j] = 1.0
_B16 = np.zeros((16, 128), np.float32)
for _i in range(9):
    for _j in range(8):
        _B16[_j, _i * 8 + _j] = 1.0
_RH = np.zeros((16, 128), np.float32)
for _m in range(16):
    for _k in range(8):
        _RH[_m, _m * 8 + _k] = 1.0
_S = np.zeros((128, 8), np.float32)
for _m in range(16):
    for _k in range(8):
        _S[_m * 8 + _k, _k] = 1.0


def _sc_gather(table, idx_all):
    mesh = plsc.VectorSubcoreMesh(core_axis_name="c", subcore_axis_name="s")
    n_idx = idx_all.shape[0]

    nbuf = 4

    @functools.partial(
        pl.kernel,
        out_type=jax.ShapeDtypeStruct((n_idx, 16), jnp.float32),
        mesh=mesh,
        scratch_types=[
            pltpu.VMEM((CPW * CH,), jnp.int32),
        ] + [pltpu.VMEM((CH, 16), jnp.float32) for _ in range(nbuf)]
          + [pltpu.SemaphoreType.DMA for _ in range(2 * nbuf)],
        compiler_params=pltpu.CompilerParams(use_tc_tiling_on_sc=False),
    )
    def k(table_hbm, idx_hbm, out_hbm, idx_v, r0, r1, r2, r3,
          g0, g1, g2, g3, o0, o1, o2, o3):
        rows = (r0, r1, r2, r3)
        gsem = (g0, g1, g2, g3)
        osem = (o0, o1, o2, o3)
        wid = lax.axis_index("s") * 2 + lax.axis_index("c")
        base = pl.multiple_of(wid * (CPW * CH), CH)
        pltpu.sync_copy(idx_hbm.at[pl.ds(base, CPW * CH)], idx_v)

        def body(j, carry):
            hs = []
            for b in range(nbuf):
                @pl.when(j > 0)
                def _(b=b):
                    pltpu.make_async_copy(
                        rows[b], out_hbm.at[pl.ds(0, CH)], osem[b]).wait()
                hs.append(pltpu.async_copy(
                    table_hbm.at[idx_v.at[pl.ds((nbuf * j + b) * CH, CH)]],
                    rows[b], gsem[b]))
            for b in range(nbuf):
                hs[b].wait()
                off = pl.multiple_of(base + (nbuf * j + b) * CH, CH)
                pltpu.async_copy(rows[b], out_hbm.at[pl.ds(off, CH)], osem[b])
            return carry

        lax.fori_loop(0, CPW // nbuf, body, 0)
        for b in range(nbuf):
            pltpu.make_async_copy(
                rows[b], out_hbm.at[pl.ds(0, CH)], osem[b]).wait()

    return k(table, idx_all)


def _tc_body(gl_ref, gr_ref, w1b_ref, w2_ref, a_ref, b_ref, s_ref, out_ref):
    gl = gl_ref[...]
    gr = gr_ref[...]
    lane = lax.broadcasted_iota(jnp.int32, (EB, 16), 1)

    d = gl - gr
    d2 = d * d
    r2 = jnp.sum(jnp.where((lane >= 8) & (lane < 11), d2, 0.0), axis=1,
                 keepdims=True)
    t = jnp.sqrt((r2 + 1e-12) * _ISTEP2)
    diff = t - (lane.astype(jnp.float32) + 1.0)
    q = diff * diff
    den = 1.0 - q
    emb = jnp.where(q < 1.0, jnp.exp(-2.0 / den), 0.0)

    dot = functools.partial(jnp.dot, precision=lax.Precision.HIGHEST,
                            preferred_element_type=jnp.float32)
    yb = dot(emb, w1b_ref[...])
    hb = ACT_NORM * jax.nn.silu(yb)

    xb1 = dot(gl, a_ref[...])
    xb2 = dot(gr, b_ref[...])
    g = xb1 * xb2

    p = dot(g, w2_ref[...])
    out_ref[...] = dot(p * hb, s_ref[...])


def _tc_compute(gl, gr, w1b, w2p, a16, b16, s):
    grid = E_PAD // EB
    full = lambda i: (0, 0)
    tile = lambda i: (i, 0)
    return pl.pallas_call(
        _tc_body,
        grid=(grid,),
        in_specs=[
            pl.BlockSpec((EB, 16), tile),
            pl.BlockSpec((EB, 16), tile),
            pl.BlockSpec((16, 128), full),
            pl.BlockSpec((128, 128), full),
            pl.BlockSpec((16, 128), full),
            pl.BlockSpec((16, 128), full),
            pl.BlockSpec((128, 8), full),
        ],
        out_specs=pl.BlockSpec((EB, 8), tile),
        out_shape=jax.ShapeDtypeStruct((E_PAD, 8), jnp.float32),
    )(gl, gr, w1b, w2p, a16, b16, s)


def kernel(x, pos, edge_index, W1, W2):
    idx = edge_index.astype(jnp.int32)
    table = jnp.concatenate(
        [x, pos, jnp.ones((N_NODES, 1), x.dtype),
         jnp.zeros((N_NODES, 4), x.dtype)], axis=1)
    pad = E_PAD - N_EDGES
    lig = jnp.pad(idx[1], (0, pad))
    rec = jnp.pad(idx[0], (0, pad))
    idx_all = jnp.concatenate([lig, rec])

    g = _sc_gather(table, idx_all)
    gl, gr = g[:E_PAD], g[E_PAD:]

    w1p = jnp.concatenate(
        [W1, jnp.zeros((6, 16), W1.dtype)], axis=0) * (_EMB_C / np.sqrt(NB))
    w1b = w1p @ jnp.asarray(_RH)
    w2r = W2.reshape(16, 9, 8, 8).transpose(1, 2, 0, 3).reshape(72, 128)
    w2p = jnp.concatenate(
        [w2r, jnp.zeros((56, 128), W2.dtype)], axis=0) / (4.0 * np.sqrt(72.0))

    out = _tc_compute(gl, gr, w1b, w2p,
                      jnp.asarray(_A16), jnp.asarray(_B16), jnp.asarray(_S))
    return out[:N_EDGES]


# speedup vs baseline: 2.3276x; 1.2131x over previous
import functools

import jax
import jax.numpy as jnp
import numpy as np
from jax import lax
from jax.experimental import pallas as pl
from jax.experimental.pallas import tpu as pltpu
from jax.experimental.pallas import tpu_sc as plsc

N_NODES = 10000
N_EDGES = 160000
RADIUS = 5.0
NB = 10
ACT_NORM = 1.6790

NW = 32
CH = 128
CPW = 80
E_PAD = 163840
EB = 2000

_STEP = RADIUS / (NB + 1)
_ISTEP2 = 1.0 / (_STEP * _STEP)
_EMB_C = 1.14136 * float(np.exp(2.0)) * float(np.sqrt(NB))

_A16 = np.zeros((16, 128), np.float32)
for _i in range(8):
    for _j in range(8):
        _A16[_i, _i * 8 + _j] = 1.0
for _j in range(8):
    _A16[11, 64 + _j] = 1.0
_B16 = np.zeros((16, 128), np.float32)
for _i in range(9):
    for _j in range(8):
        _B16[_j, _i * 8 + _j] = 1.0
_RH = np.zeros((16, 128), np.float32)
for _m in range(16):
    for _k in range(8):
        _RH[_m, _m * 8 + _k] = 1.0
_S = np.zeros((128, 8), np.float32)
for _m in range(16):
    for _k in range(8):
        _S[_m * 8 + _k, _k] = 1.0


def _sc_gather(table, idx_all):
    mesh = plsc.VectorSubcoreMesh(core_axis_name="c", subcore_axis_name="s")
    n_idx = idx_all.shape[0]

    nbuf = 4
    ipw = E_PAD // NW
    cpw = ipw // CH

    @functools.partial(
        pl.kernel,
        out_type=[jax.ShapeDtypeStruct((E_PAD, 16), jnp.float32),
                  jax.ShapeDtypeStruct((E_PAD, 16), jnp.float32)],
        mesh=mesh,
        scratch_types=[
            pltpu.VMEM((2 * ipw,), jnp.int32),
        ] + [pltpu.VMEM((CH, 16), jnp.float32) for _ in range(nbuf)]
          + [pltpu.SemaphoreType.DMA for _ in range(2 * nbuf)],
        compiler_params=pltpu.CompilerParams(use_tc_tiling_on_sc=False),
    )
    def k(table_hbm, idx_hbm, outl_hbm, outr_hbm, idx_v, r0, r1, r2, r3,
          g0, g1, g2, g3, o0, o1, o2, o3):
        rows = (r0, r1, r2, r3)
        gsem = (g0, g1, g2, g3)
        osem = (o0, o1, o2, o3)
        outs = (outl_hbm, outl_hbm, outr_hbm, outr_hbm)
        wid = lax.axis_index("s") * 2 + lax.axis_index("c")
        base = pl.multiple_of(wid * ipw, CH)
        pltpu.sync_copy(idx_hbm.at[pl.ds(base, ipw)], idx_v.at[pl.ds(0, ipw)])
        pltpu.sync_copy(idx_hbm.at[pl.ds(E_PAD + base, ipw)],
                        idx_v.at[pl.ds(ipw, ipw)])

        def body(j, carry):
            hs = []
            for b in range(nbuf):
                voff = (b // 2) * ipw + (2 * j + (b % 2)) * CH
                @pl.when(j > 0)
                def _(b=b):
                    pltpu.make_async_copy(
                        rows[b], outl_hbm.at[pl.ds(0, CH)], osem[b]).wait()
                hs.append(pltpu.async_copy(
                    table_hbm.at[idx_v.at[pl.ds(voff, CH)]],
                    rows[b], gsem[b]))
            for b in range(nbuf):
                hs[b].wait()
                off = pl.multiple_of(base + (2 * j + (b % 2)) * CH, CH)
                pltpu.async_copy(rows[b], outs[b].at[pl.ds(off, CH)], osem[b])
            return carry

        lax.fori_loop(0, cpw // 2, body, 0)
        for b in range(nbuf):
            pltpu.make_async_copy(
                rows[b], outl_hbm.at[pl.ds(0, CH)], osem[b]).wait()

    return k(table, idx_all)


def _tc_body(gl_ref, gr_ref, w1b_ref, w2_ref, a_ref, b_ref, s_ref, out_ref):
    gl = gl_ref[...]
    gr = gr_ref[...]
    lane = lax.broadcasted_iota(jnp.int32, (EB, 16), 1)

    d = gl - gr
    d2 = d * d
    r2 = jnp.sum(jnp.where((lane >= 8) & (lane < 11), d2, 0.0), axis=1,
                 keepdims=True)
    t = jnp.sqrt((r2 + 1e-12) * _ISTEP2)
    diff = t - (lane.astype(jnp.float32) + 1.0)
    q = diff * diff
    den = 1.0 - q
    emb = jnp.where(q < 1.0, jnp.exp(-2.0 / den), 0.0)

    dot = functools.partial(jnp.dot, precision=lax.Precision.HIGHEST,
                            preferred_element_type=jnp.float32)
    yb = dot(emb, w1b_ref[...])
    hb = ACT_NORM * jax.nn.silu(yb)

    xb1 = dot(gl, a_ref[...])
    xb2 = dot(gr, b_ref[...])
    g = xb1 * xb2

    p = dot(g, w2_ref[...])
    out_ref[...] = dot(p * hb, s_ref[...])


def _tc_compute(gl, gr, w1b, w2p, a16, b16, s):
    grid = N_EDGES // EB
    full = lambda i: (0, 0)
    tile = lambda i: (i, 0)
    return pl.pallas_call(
        _tc_body,
        grid=(grid,),
        in_specs=[
            pl.BlockSpec((EB, 16), tile),
            pl.BlockSpec((EB, 16), tile),
            pl.BlockSpec((16, 128), full),
            pl.BlockSpec((128, 128), full),
            pl.BlockSpec((16, 128), full),
            pl.BlockSpec((16, 128), full),
            pl.BlockSpec((128, 8), full),
        ],
        out_specs=pl.BlockSpec((EB, 8), tile),
        out_shape=jax.ShapeDtypeStruct((N_EDGES, 8), jnp.float32),
    )(gl, gr, w1b, w2p, a16, b16, s)


def kernel(x, pos, edge_index, W1, W2):
    idx = edge_index.astype(jnp.int32)
    table = jnp.concatenate(
        [x, pos, jnp.ones((N_NODES, 1), x.dtype),
         jnp.zeros((N_NODES, 4), x.dtype)], axis=1)
    pad = E_PAD - N_EDGES
    lig = jnp.pad(idx[1], (0, pad))
    rec = jnp.pad(idx[0], (0, pad))
    idx_all = jnp.concatenate([lig, rec])

    gl, gr = _sc_gather(table, idx_all)

    w1p = jnp.concatenate(
        [W1, jnp.zeros((6, 16), W1.dtype)], axis=0) * (_EMB_C / np.sqrt(NB))
    w1b = w1p @ jnp.asarray(_RH)
    w2r = W2.reshape(16, 9, 8, 8).transpose(1, 2, 0, 3).reshape(72, 128)
    w2p = jnp.concatenate(
        [w2r, jnp.zeros((56, 128), W2.dtype)], axis=0) / (4.0 * np.sqrt(72.0))

    return _tc_compute(gl, gr, w1b, w2p,
                       jnp.asarray(_A16), jnp.asarray(_B16), jnp.asarray(_S))


# speedup vs baseline: 4.8771x; 2.0954x over previous
import functools

import jax
import jax.numpy as jnp
import numpy as np
from jax import lax
from jax.experimental import pallas as pl
from jax.experimental.pallas import tpu as pltpu
from jax.experimental.pallas import tpu_sc as plsc

N_NODES = 10000
N_EDGES = 160000
RADIUS = 5.0
NB = 10
ACT_NORM = 1.6790

NW = 32
CH = 128
CPW = 80
E_PAD = 163840
EB = 4000

_STEP = RADIUS / (NB + 1)
_ISTEP2 = 1.0 / (_STEP * _STEP)
_EMB_C = 1.14136 * float(np.exp(2.0)) * float(np.sqrt(NB))

_A16 = np.zeros((16, 128), np.float32)
for _i in range(8):
    for _j in range(8):
        _A16[_i, _i * 8 + _j] = 1.0
for _j in range(8):
    _A16[11, 64 + _j] = 1.0
_B16 = np.zeros((16, 128), np.float32)
for _i in range(9):
    for _j in range(8):
        _B16[_j, _i * 8 + _j] = 1.0
_RH = np.zeros((16, 128), np.float32)
for _m in range(16):
    for _k in range(8):
        _RH[_m, _m * 8 + _k] = 1.0
_S = np.zeros((128, 8), np.float32)
for _m in range(16):
    for _k in range(8):
        _S[_m * 8 + _k, _k] = 1.0


def _sc_gather(table, idx_all):
    mesh = plsc.VectorSubcoreMesh(core_axis_name="c", subcore_axis_name="s")
    n_idx = idx_all.shape[0]

    nbuf = 4
    ipw = E_PAD // NW
    cpw = ipw // CH

    @functools.partial(
        pl.kernel,
        out_type=[jax.ShapeDtypeStruct((E_PAD, 16), jnp.float32),
                  jax.ShapeDtypeStruct((E_PAD, 16), jnp.float32)],
        mesh=mesh,
        scratch_types=[
            pltpu.VMEM((2 * ipw,), jnp.int32),
        ] + [pltpu.VMEM((CH, 16), jnp.float32) for _ in range(nbuf)]
          + [pltpu.SemaphoreType.DMA for _ in range(2 * nbuf)],
        compiler_params=pltpu.CompilerParams(use_tc_tiling_on_sc=False),
    )
    def k(table_hbm, idx_hbm, outl_hbm, outr_hbm, idx_v, r0, r1, r2, r3,
          g0, g1, g2, g3, o0, o1, o2, o3):
        rows = (r0, r1, r2, r3)
        gsem = (g0, g1, g2, g3)
        osem = (o0, o1, o2, o3)
        outs = (outl_hbm, outl_hbm, outr_hbm, outr_hbm)
        wid = lax.axis_index("s") * 2 + lax.axis_index("c")
        base = pl.multiple_of(wid * ipw, CH)
        pltpu.sync_copy(idx_hbm.at[pl.ds(base, ipw)], idx_v.at[pl.ds(0, ipw)])
        pltpu.sync_copy(idx_hbm.at[pl.ds(E_PAD + base, ipw)],
                        idx_v.at[pl.ds(ipw, ipw)])

        def body(j, carry):
            hs = []
            for b in range(nbuf):
                voff = (b // 2) * ipw + (2 * j + (b % 2)) * CH
                @pl.when(j > 0)
                def _(b=b):
                    pltpu.make_async_copy(
                        rows[b], outl_hbm.at[pl.ds(0, CH)], osem[b]).wait()
                hs.append(pltpu.async_copy(
                    table_hbm.at[idx_v.at[pl.ds(voff, CH)]],
                    rows[b], gsem[b]))
            for b in range(nbuf):
                hs[b].wait()
                off = pl.multiple_of(base + (2 * j + (b % 2)) * CH, CH)
                pltpu.async_copy(rows[b], outs[b].at[pl.ds(off, CH)], osem[b])
            return carry

        lax.fori_loop(0, cpw // 2, body, 0)
        for b in range(nbuf):
            pltpu.make_async_copy(
                rows[b], outl_hbm.at[pl.ds(0, CH)], osem[b]).wait()

    return k(table, idx_all)


def _tc_body(gl_ref, gr_ref, w1b_ref, w2_ref, a_ref, b_ref, s_ref, out_ref):
    gl = gl_ref[...]
    gr = gr_ref[...]
    lane = lax.broadcasted_iota(jnp.int32, (EB, 16), 1)

    d = gl - gr
    d2 = d * d
    r2 = jnp.sum(jnp.where((lane >= 8) & (lane < 11), d2, 0.0), axis=1,
                 keepdims=True)
    t = jnp.sqrt((r2 + 1e-12) * _ISTEP2)
    diff = t - (lane.astype(jnp.float32) + 1.0)
    q = diff * diff
    den = 1.0 - q
    emb = jnp.where(q < 1.0, jnp.exp(-2.0 / den), 0.0)

    dot = functools.partial(jnp.dot, precision=lax.Precision.DEFAULT,
                            preferred_element_type=jnp.float32)
    yb = dot(emb, w1b_ref[...])
    hb = ACT_NORM * jax.nn.silu(yb)

    xb1 = dot(gl, a_ref[...])
    xb2 = dot(gr, b_ref[...])
    g = xb1 * xb2

    p = dot(g, w2_ref[...])
    out_ref[...] = dot(p * hb, s_ref[...])


def _tc_compute(gl, gr, w1b, w2p, a16, b16, s):
    grid = N_EDGES // EB
    full = lambda i: (0, 0)
    tile = lambda i: (i, 0)
    return pl.pallas_call(
        _tc_body,
        grid=(grid,),
        in_specs=[
            pl.BlockSpec((EB, 16), tile),
            pl.BlockSpec((EB, 16), tile),
            pl.BlockSpec((16, 128), full),
            pl.BlockSpec((128, 128), full),
            pl.BlockSpec((16, 128), full),
            pl.BlockSpec((16, 128), full),
            pl.BlockSpec((128, 8), full),
        ],
        out_specs=pl.BlockSpec((EB, 8), tile),
        out_shape=jax.ShapeDtypeStruct((N_EDGES, 8), jnp.float32),
    )(gl, gr, w1b, w2p, a16, b16, s)


def kernel(x, pos, edge_index, W1, W2):
    idx = edge_index.astype(jnp.int32)
    table = jnp.concatenate(
        [x, pos, jnp.ones((N_NODES, 1), x.dtype),
         jnp.zeros((N_NODES, 4), x.dtype)], axis=1)
    pad = E_PAD - N_EDGES
    lig = jnp.pad(idx[1], (0, pad))
    rec = jnp.pad(idx[0], (0, pad))
    idx_all = jnp.concatenate([lig, rec])

    gl, gr = _sc_gather(table, idx_all)

    w1p = jnp.concatenate(
        [W1, jnp.zeros((6, 16), W1.dtype)], axis=0) * (_EMB_C / np.sqrt(NB))
    w1b = w1p @ jnp.asarray(_RH)
    w2r = W2.reshape(16, 9, 8, 8).transpose(1, 2, 0, 3).reshape(72, 128)
    w2p = jnp.concatenate(
        [w2r, jnp.zeros((56, 128), W2.dtype)], axis=0) / (4.0 * np.sqrt(72.0))

    return _tc_compute(gl, gr, w1b, w2p,
                       jnp.asarray(_A16), jnp.asarray(_B16), jnp.asarray(_S))


# speedup vs baseline: 6.0716x; 1.2449x over previous
import functools

import jax
import jax.numpy as jnp
import numpy as np
from jax import lax
from jax.experimental import pallas as pl
from jax.experimental.pallas import tpu as pltpu
from jax.experimental.pallas import tpu_sc as plsc

N_NODES = 10000
N_EDGES = 160000
RADIUS = 5.0
NB = 10
ACT_NORM = 1.6790

NW = 32
CH = 128
CPW = 80
E_PAD = 163840
EB = 3200

_STEP = RADIUS / (NB + 1)
_ISTEP2 = 1.0 / (_STEP * _STEP)
_EMB_C = 1.14136 * float(np.exp(2.0)) * float(np.sqrt(NB))

_A16 = np.zeros((16, 128), np.float32)
for _i in range(8):
    for _j in range(8):
        _A16[_i, _i * 8 + _j] = 1.0
for _j in range(8):
    _A16[11, 64 + _j] = 1.0
_B16 = np.zeros((16, 128), np.float32)
for _i in range(9):
    for _j in range(8):
        _B16[_j, _i * 8 + _j] = 1.0
_RH = np.zeros((16, 128), np.float32)
for _m in range(16):
    for _k in range(8):
        _RH[_m, _m * 8 + _k] = 1.0
_S = np.zeros((128, 8), np.float32)
for _m in range(16):
    for _k in range(8):
        _S[_m * 8 + _k, _k] = 1.0
_SP = np.zeros((128, 128), np.float32)
for _t in range(8):
    for _c in (8, 9, 10):
        for _c2 in range(16):
            _SP[16 * _t + _c, 16 * _t + _c2] = 1.0
_ABLK = np.zeros((8, 128, 128), np.float32)
_BBLK = np.zeros((8, 128, 128), np.float32)
_PLACE = np.zeros((8, 128, 16), np.float32)
for _t in range(8):
    _ABLK[_t, 16 * _t:16 * _t + 16, :] = _A16
    _BBLK[_t, 16 * _t:16 * _t + 16, :] = _B16
    for _a in range(16):
        _PLACE[_t, 16 * _t + _a, _a] = 1.0


def _sc_gather(table, idx_all):
    mesh = plsc.VectorSubcoreMesh(core_axis_name="c", subcore_axis_name="s")
    n_idx = idx_all.shape[0]

    nbuf = 4
    ipw = E_PAD // NW
    cpw = ipw // CH

    @functools.partial(
        pl.kernel,
        out_type=[jax.ShapeDtypeStruct((E_PAD, 16), jnp.float32),
                  jax.ShapeDtypeStruct((E_PAD, 16), jnp.float32)],
        mesh=mesh,
        scratch_types=[
            pltpu.VMEM((2 * ipw,), jnp.int32),
        ] + [pltpu.VMEM((CH, 16), jnp.float32) for _ in range(nbuf)]
          + [pltpu.SemaphoreType.DMA for _ in range(2 * nbuf)],
        compiler_params=pltpu.CompilerParams(use_tc_tiling_on_sc=False),
    )
    def k(table_hbm, idx_hbm, outl_hbm, outr_hbm, idx_v, r0, r1, r2, r3,
          g0, g1, g2, g3, o0, o1, o2, o3):
        rows = (r0, r1, r2, r3)
        gsem = (g0, g1, g2, g3)
        osem = (o0, o1, o2, o3)
        outs = (outl_hbm, outl_hbm, outr_hbm, outr_hbm)
        wid = lax.axis_index("s") * 2 + lax.axis_index("c")
        base = pl.multiple_of(wid * ipw, CH)
        pltpu.sync_copy(idx_hbm.at[pl.ds(base, ipw)], idx_v.at[pl.ds(0, ipw)])
        pltpu.sync_copy(idx_hbm.at[pl.ds(E_PAD + base, ipw)],
                        idx_v.at[pl.ds(ipw, ipw)])

        def body(j, carry):
            hs = []
            for b in range(nbuf):
                voff = (b // 2) * ipw + (2 * j + (b % 2)) * CH
                @pl.when(j > 0)
                def _(b=b):
                    pltpu.make_async_copy(
                        rows[b], outl_hbm.at[pl.ds(0, CH)], osem[b]).wait()
                hs.append(pltpu.async_copy(
                    table_hbm.at[idx_v.at[pl.ds(voff, CH)]],
                    rows[b], gsem[b]))
            for b in range(nbuf):
                hs[b].wait()
                off = pl.multiple_of(base + (2 * j + (b % 2)) * CH, CH)
                pltpu.async_copy(rows[b], outs[b].at[pl.ds(off, CH)], osem[b])
            return carry

        lax.fori_loop(0, cpw // 2, body, 0)
        for b in range(nbuf):
            pltpu.make_async_copy(
                rows[b], outl_hbm.at[pl.ds(0, CH)], osem[b]).wait()

    return k(table, idx_all)


def _tc_body(glp_ref, grp_ref, w1blk_ref, w2_ref, ablk_ref, bblk_ref,
             sp_ref, s_ref, out_ref):
    glp = glp_ref[...]
    grp = grp_ref[...]
    dot = functools.partial(jnp.dot, precision=lax.Precision.DEFAULT,
                            preferred_element_type=jnp.float32)

    d = glp - grp
    d2 = d * d
    r2b = jnp.dot(d2, sp_ref[...], precision=lax.Precision.HIGHEST,
                  preferred_element_type=jnp.float32)
    t = jnp.sqrt((r2b + 1e-12) * _ISTEP2)
    lane = lax.broadcasted_iota(jnp.int32, (EB // 8, 128), 1)
    c1 = ((lane & 15) + 1).astype(jnp.float32)
    diff = t - c1
    q = diff * diff
    den = 1.0 - q
    embp = jnp.where(q < 1.0, jnp.exp(-2.0 / den), 0.0)

    yb = jnp.concatenate(
        [dot(embp, w1blk_ref[t2]) for t2 in range(8)], axis=0)
    hb = ACT_NORM * jax.nn.silu(yb)

    xb1 = jnp.concatenate(
        [dot(glp, ablk_ref[t2]) for t2 in range(8)], axis=0)
    xb2 = jnp.concatenate(
        [dot(grp, bblk_ref[t2]) for t2 in range(8)], axis=0)
    g = xb1 * xb2

    p = dot(g, w2_ref[...])
    out_ref[...] = dot(p * hb, s_ref[...])


def _tc_compute(glp, grp, w1blk, w2p, ablk, bblk, sp, sel):
    grid = N_EDGES // EB
    full2 = lambda i: (0, 0)
    full3 = lambda i: (0, 0, 0)
    tile = lambda i: (i, 0)
    return pl.pallas_call(
        _tc_body,
        grid=(grid,),
        in_specs=[
            pl.BlockSpec((EB // 8, 128), tile),
            pl.BlockSpec((EB // 8, 128), tile),
            pl.BlockSpec((8, 128, 128), full3),
            pl.BlockSpec((128, 128), full2),
            pl.BlockSpec((8, 128, 128), full3),
            pl.BlockSpec((8, 128, 128), full3),
            pl.BlockSpec((128, 128), full2),
            pl.BlockSpec((128, 8), full2),
        ],
        out_specs=pl.BlockSpec((EB, 8), tile),
        out_shape=jax.ShapeDtypeStruct((N_EDGES, 8), jnp.float32),
    )(glp, grp, w1blk, w2p, ablk, bblk, sp, sel)


def kernel(x, pos, edge_index, W1, W2):
    idx = edge_index.astype(jnp.int32)
    idx = idx.reshape(2, N_EDGES // EB, 8, EB // 8)
    idx = idx.transpose(0, 1, 3, 2).reshape(2, N_EDGES)
    table = jnp.concatenate(
        [x, pos, jnp.ones((N_NODES, 1), x.dtype),
         jnp.zeros((N_NODES, 4), x.dtype)], axis=1)
    pad = E_PAD - N_EDGES
    lig = jnp.pad(idx[1], (0, pad))
    rec = jnp.pad(idx[0], (0, pad))
    idx_all = jnp.concatenate([lig, rec])

    gl, gr = _sc_gather(table, idx_all)

    w1p = jnp.concatenate(
        [W1, jnp.zeros((6, 16), W1.dtype)], axis=0) * (_EMB_C / np.sqrt(NB))
    w1b = w1p @ jnp.asarray(_RH)
    w1blk = jnp.asarray(_PLACE) @ w1b
    w2r = W2.reshape(16, 9, 8, 8).transpose(1, 2, 0, 3).reshape(72, 128)
    w2p = jnp.concatenate(
        [w2r, jnp.zeros((56, 128), W2.dtype)], axis=0) / (4.0 * np.sqrt(72.0))

    glp = gl.reshape(E_PAD // 8, 128)
    grp = gr.reshape(E_PAD // 8, 128)
    return _tc_compute(glp, grp, w1blk, w2p,
                       jnp.asarray(_ABLK), jnp.asarray(_BBLK),
                       jnp.asarray(_SP), jnp.asarray(_S))
